# Initial kernel scaffold; baseline (speedup 1.0000x reference)
import jax
import jax.numpy as jnp
from jax.experimental import pallas as pl


def kernel(x, edge_index, W1, b1, W2, b2):
    raise NotImplementedError("write your pallas kernel here")


# speedup vs baseline: 15.1122x; 15.1122x over previous
import functools

import jax
import jax.numpy as jnp
from jax import lax
from jax.experimental import pallas as pl
from jax.experimental.pallas import tpu as pltpu
from jax.experimental.pallas import tpu_sc as plsc

N = 10000
D = 128
E = 320000
NC, NS, NW = 2, 16, 32
EPW = E // NW
CH = 128
NFULL = EPW // CH
REM = EPW - NFULL * CH
WR = 624
WR_LAST = N - 15 * WR
NP = 10240


@functools.cache
def _sc_kernels():
    mesh = plsc.VectorSubcoreMesh(core_axis_name="c", subcore_axis_name="s",
                                  num_cores=NC, num_subcores=NS)

    @functools.partial(
        pl.kernel,
        out_type=jax.ShapeDtypeStruct((NC * N, D), jnp.float32),
        mesh=mesh,
        scratch_types=[
            pltpu.VMEM((CH,), jnp.int32),
            pltpu.VMEM((REM,), jnp.int32),
            pltpu.VMEM((CH, D), jnp.float32),
            pltpu.VMEM_SHARED((N, D), jnp.float32),
        ],
    )
    def deg_kernel(dst_hbm, zeros_hbm, ones_hbm, out_hbm,
                   idx_v, idxr_v, ones_v, acc):
        c = lax.axis_index("c")
        s = lax.axis_index("s")
        base = c * (E // NC) + s * EPW

        @pl.when(s < NS - 1)
        def _zero():
            pltpu.sync_copy(zeros_hbm.at[pl.ds(0, WR)],
                            acc.at[pl.ds(s * WR, WR)])

        @pl.when(s == NS - 1)
        def _zero_last():
            pltpu.sync_copy(zeros_hbm, acc.at[pl.ds(15 * WR, WR_LAST)])

        pltpu.sync_copy(ones_hbm, ones_v)
        plsc.subcore_barrier()

        @pl.loop(0, NFULL)
        def _chunk(i):
            pltpu.sync_copy(dst_hbm.at[pl.ds(base + i * CH, CH)], idx_v)
            pltpu.sync_copy(ones_v, acc.at[idx_v], add=True)

        pltpu.sync_copy(dst_hbm.at[pl.ds(base + NFULL * CH, REM)], idxr_v)
        pltpu.sync_copy(ones_v.at[pl.ds(0, REM)], acc.at[idxr_v], add=True)
        plsc.subcore_barrier()

        @pl.when(s < NS - 1)
        def _wout():
            pltpu.sync_copy(acc.at[pl.ds(s * WR, WR)],
                            out_hbm.at[pl.ds(c * N + s * WR, WR)])

        @pl.when(s == NS - 1)
        def _wout_last():
            pltpu.sync_copy(acc.at[pl.ds(15 * WR, WR_LAST)],
                            out_hbm.at[pl.ds(c * N + 15 * WR, WR_LAST)])

    @functools.partial(
        pl.kernel,
        out_type=jax.ShapeDtypeStruct((NC * N, D), jnp.float32),
        mesh=mesh,
        scratch_types=[
            pltpu.VMEM((CH,), jnp.int32),
            pltpu.VMEM((CH,), jnp.int32),
            pltpu.VMEM((REM,), jnp.int32),
            pltpu.VMEM((REM,), jnp.int32),
            pltpu.VMEM((CH, D), jnp.float32),
            pltpu.VMEM((REM, D), jnp.float32),
            pltpu.VMEM_SHARED((N, D), jnp.float32),
            pltpu.SemaphoreType.DMA,
        ],
    )
    def scatter_kernel(g_hbm, src_hbm, dst_hbm, zeros_hbm, out_hbm,
                       idx_s, idx_d, idx_sr, idx_dr, rows, rows_r, acc, gsem):
        c = lax.axis_index("c")
        s = lax.axis_index("s")
        base = c * (E // NC) + s * EPW

        @pl.when(s < NS - 1)
        def _zero():
            pltpu.sync_copy(zeros_hbm.at[pl.ds(0, WR)],
                            acc.at[pl.ds(s * WR, WR)])

        @pl.when(s == NS - 1)
        def _zero_last():
            pltpu.sync_copy(zeros_hbm, acc.at[pl.ds(15 * WR, WR_LAST)])

        plsc.subcore_barrier()

        @pl.loop(0, NFULL)
        def _chunk(i):
            off = base + i * CH
            pltpu.sync_copy(src_hbm.at[pl.ds(off, CH)], idx_s)
            pltpu.sync_copy(dst_hbm.at[pl.ds(off, CH)], idx_d)
            pltpu.async_copy(g_hbm.at[idx_s], rows, gsem).wait()
            pltpu.sync_copy(rows, acc.at[idx_d], add=True)

        off = base + NFULL * CH
        pltpu.sync_copy(src_hbm.at[pl.ds(off, REM)], idx_sr)
        pltpu.sync_copy(dst_hbm.at[pl.ds(off, REM)], idx_dr)
        pltpu.async_copy(g_hbm.at[idx_sr], rows_r, gsem).wait()
        pltpu.sync_copy(rows_r, acc.at[idx_dr], add=True)
        plsc.subcore_barrier()

        @pl.when(s < NS - 1)
        def _wout():
            pltpu.sync_copy(acc.at[pl.ds(s * WR, WR)],
                            out_hbm.at[pl.ds(c * N + s * WR, WR)])

        @pl.when(s == NS - 1)
        def _wout_last():
            pltpu.sync_copy(acc.at[pl.ds(15 * WR, WR_LAST)],
                            out_hbm.at[pl.ds(c * N + 15 * WR, WR_LAST)])

    return deg_kernel, scatter_kernel


BLK = 2000


def _dinv_of(dp):
    return lax.rsqrt(dp[0, :, 0:1] + dp[1, :, 0:1] + 1.0)


def _m1_body(x_ref, w_ref, dp_ref, g_ref):
    dinv = _dinv_of(dp_ref[...])
    g_ref[...] = dinv * jnp.dot(x_ref[...], w_ref[...],
                                preferred_element_type=jnp.float32)


def _m2_body(p_ref, g1_ref, dp_ref, b_ref, w_ref, g2_ref):
    dinv = _dinv_of(dp_ref[...])
    p = p_ref[...]
    z = jnp.maximum(dinv * (p[0] + p[1] + g1_ref[...]) + b_ref[...], 0.0)
    g2_ref[...] = dinv * jnp.dot(z, w_ref[...],
                                 preferred_element_type=jnp.float32)


def _m3_body(p_ref, g2_ref, dp_ref, b_ref, o_ref):
    dinv = _dinv_of(dp_ref[...])
    p = p_ref[...]
    o_ref[...] = dinv * (p[0] + p[1] + g2_ref[...]) + b_ref[...]


def _row_spec(blk=BLK):
    return pl.BlockSpec((blk, D), lambda i: (i, 0))


def _m1_call(x, W1, degp):
    return pl.pallas_call(
        _m1_body,
        grid=(N // BLK,),
        in_specs=[
            _row_spec(),
            pl.BlockSpec((D, D), lambda i: (0, 0)),
            pl.BlockSpec((2, BLK, 16), lambda i: (0, i, 0)),
        ],
        out_specs=_row_spec(),
        out_shape=jax.ShapeDtypeStruct((N, D), jnp.float32),
    )(x, W1, degp)


def _m2_call(P1, g1, degp, b1, W2):
    return pl.pallas_call(
        _m2_body,
        grid=(N // BLK,),
        in_specs=[
            pl.BlockSpec((2, BLK, D), lambda i: (0, i, 0)),
            _row_spec(),
            pl.BlockSpec((2, BLK, 16), lambda i: (0, i, 0)),
            pl.BlockSpec((1, D), lambda i: (0, 0)),
            pl.BlockSpec((D, D), lambda i: (0, 0)),
        ],
        out_specs=_row_spec(),
        out_shape=jax.ShapeDtypeStruct((N, D), jnp.float32),
    )(P1, g1, degp, b1, W2)


def _m3_call(P2, g2, degp, b2):
    return pl.pallas_call(
        _m3_body,
        grid=(N // BLK,),
        in_specs=[
            pl.BlockSpec((2, BLK, D), lambda i: (0, i, 0)),
            _row_spec(),
            pl.BlockSpec((2, BLK, 16), lambda i: (0, i, 0)),
            pl.BlockSpec((1, D), lambda i: (0, 0)),
        ],
        out_specs=_row_spec(),
        out_shape=jax.ShapeDtypeStruct((N, D), jnp.float32),
    )(P2, g2, degp, b2)


def kernel(x, edge_index, W1, b1, W2, b2):
    deg_kernel, scatter_kernel = _sc_kernels()
    src = edge_index[0]
    dst = edge_index[1]
    zerosD = jnp.zeros((WR_LAST, D), jnp.float32)
    onesD = jnp.ones((CH, D), jnp.float32)

    degp = deg_kernel(dst, zerosD, onesD).reshape(NC, N, D)[:, :, :16]
    g1 = _m1_call(x, W1, degp)
    P1 = scatter_kernel(g1, src, dst, zerosD).reshape(NC, N, D)
    g2 = _m2_call(P1, g1, degp, b1.reshape(1, D), W2)
    P2 = scatter_kernel(g2, src, dst, zerosD).reshape(NC, N, D)
    out = _m3_call(P2, g2, degp, b2.reshape(1, D))
    return out


# speedup vs baseline: 23.8049x; 1.5752x over previous
import functools

import jax
import jax.numpy as jnp
from jax import lax
from jax.experimental import pallas as pl
from jax.experimental.pallas import tpu as pltpu
from jax.experimental.pallas import tpu_sc as plsc

N = 10000
D = 128
E = 320000
NC, NS, NW = 2, 16, 32
EPW = E // NW
CH = 128
NFULL = EPW // CH
REM = EPW - NFULL * CH
WR = 624
WR_LAST = N - 15 * WR
NP = 10240


@functools.cache
def _sc_kernels():
    mesh = plsc.VectorSubcoreMesh(core_axis_name="c", subcore_axis_name="s",
                                  num_cores=NC, num_subcores=NS)

    @functools.partial(
        pl.kernel,
        out_type=jax.ShapeDtypeStruct((NC * N, D), jnp.float32),
        mesh=mesh,
        scratch_types=[
            pltpu.VMEM((CH,), jnp.int32),
            pltpu.VMEM((REM,), jnp.int32),
            pltpu.VMEM((CH, D), jnp.float32),
            pltpu.VMEM_SHARED((N, D), jnp.float32),
        ],
    )
    def deg_kernel(dst_hbm, zeros_hbm, ones_hbm, out_hbm,
                   idx_v, idxr_v, ones_v, acc):
        c = lax.axis_index("c")
        s = lax.axis_index("s")
        base = c * (E // NC) + s * EPW

        @pl.when(s < NS - 1)
        def _zero():
            pltpu.sync_copy(zeros_hbm.at[pl.ds(0, WR)],
                            acc.at[pl.ds(s * WR, WR)])

        @pl.when(s == NS - 1)
        def _zero_last():
            pltpu.sync_copy(zeros_hbm, acc.at[pl.ds(15 * WR, WR_LAST)])

        pltpu.sync_copy(ones_hbm, ones_v)
        plsc.subcore_barrier()

        @pl.loop(0, NFULL)
        def _chunk(i):
            pltpu.sync_copy(dst_hbm.at[pl.ds(base + i * CH, CH)], idx_v)
            pltpu.sync_copy(ones_v, acc.at[idx_v], add=True)

        pltpu.sync_copy(dst_hbm.at[pl.ds(base + NFULL * CH, REM)], idxr_v)
        pltpu.sync_copy(ones_v.at[pl.ds(0, REM)], acc.at[idxr_v], add=True)
        plsc.subcore_barrier()

        @pl.when(s < NS - 1)
        def _wout():
            pltpu.sync_copy(acc.at[pl.ds(s * WR, WR)],
                            out_hbm.at[pl.ds(c * N + s * WR, WR)])

        @pl.when(s == NS - 1)
        def _wout_last():
            pltpu.sync_copy(acc.at[pl.ds(15 * WR, WR_LAST)],
                            out_hbm.at[pl.ds(c * N + 15 * WR, WR_LAST)])

    NPAIR = NFULL // 2

    @functools.partial(
        pl.kernel,
        out_type=jax.ShapeDtypeStruct((NC * N, D), jnp.float32),
        mesh=mesh,
        scratch_types=[
            pltpu.VMEM((2, CH), jnp.int32),
            pltpu.VMEM((2, CH), jnp.int32),
            pltpu.VMEM((REM,), jnp.int32),
            pltpu.VMEM((REM,), jnp.int32),
            pltpu.VMEM((CH, D), jnp.float32),
            pltpu.VMEM((CH, D), jnp.float32),
            pltpu.VMEM((REM, D), jnp.float32),
            pltpu.VMEM_SHARED((N, D), jnp.float32),
            pltpu.SemaphoreType.DMA,
            pltpu.SemaphoreType.DMA,
            pltpu.SemaphoreType.DMA,
            pltpu.SemaphoreType.DMA,
        ],
    )
    def scatter_kernel(g_hbm, idx3_hbm, src_hbm, dst_hbm, zeros_hbm, out_hbm,
                       ip0, ip1, idx_sr, idx_dr, rows0, rows1, rows_r, acc,
                       gs0, gs1, ss0, ss1):
        c = lax.axis_index("c")
        s = lax.axis_index("s")
        base = c * (E // NC) + s * EPW
        cbase = (c * NS + s) * NFULL

        @pl.when(s < NS - 1)
        def _zero():
            pltpu.sync_copy(zeros_hbm.at[pl.ds(0, WR)],
                            acc.at[pl.ds(s * WR, WR)])

        @pl.when(s == NS - 1)
        def _zero_last():
            pltpu.sync_copy(zeros_hbm, acc.at[pl.ds(15 * WR, WR_LAST)])

        plsc.subcore_barrier()

        pltpu.sync_copy(idx3_hbm.at[cbase], ip0)
        pltpu.async_copy(g_hbm.at[ip0.at[0]], rows0, gs0)

        @pl.loop(0, NPAIR)
        def _pair(k):
            i = cbase + 2 * k

            @pl.when(k > 0)
            def _drain_prev_odd():
                pltpu.make_async_copy(rows1, acc.at[ip1.at[1]], ss1).wait()

            pltpu.sync_copy(idx3_hbm.at[i + 1], ip1)
            pltpu.async_copy(g_hbm.at[ip1.at[0]], rows1, gs1)
            pltpu.make_async_copy(g_hbm.at[ip0.at[0]], rows0, gs0).wait()
            pltpu.async_copy(rows0, acc.at[ip0.at[1]], ss0, add=True)

            @pl.when(k < NPAIR - 1)
            def _prefetch_even():
                pltpu.make_async_copy(rows0, acc.at[ip0.at[1]], ss0).wait()
                pltpu.sync_copy(idx3_hbm.at[i + 2], ip0)
                pltpu.async_copy(g_hbm.at[ip0.at[0]], rows0, gs0)

            pltpu.make_async_copy(g_hbm.at[ip1.at[0]], rows1, gs1).wait()
            pltpu.async_copy(rows1, acc.at[ip1.at[1]], ss1, add=True)

        pltpu.make_async_copy(rows0, acc.at[ip0.at[1]], ss0).wait()
        pltpu.make_async_copy(rows1, acc.at[ip1.at[1]], ss1).wait()

        off = base + NFULL * CH
        pltpu.sync_copy(src_hbm.at[pl.ds(off, REM)], idx_sr)
        pltpu.sync_copy(dst_hbm.at[pl.ds(off, REM)], idx_dr)
        pltpu.async_copy(g_hbm.at[idx_sr], rows_r, gs0).wait()
        pltpu.sync_copy(rows_r, acc.at[idx_dr], add=True)
        plsc.subcore_barrier()

        @pl.when(s < NS - 1)
        def _wout():
            pltpu.sync_copy(acc.at[pl.ds(s * WR, WR)],
                            out_hbm.at[pl.ds(c * N + s * WR, WR)])

        @pl.when(s == NS - 1)
        def _wout_last():
            pltpu.sync_copy(acc.at[pl.ds(15 * WR, WR_LAST)],
                            out_hbm.at[pl.ds(c * N + 15 * WR, WR_LAST)])

    return deg_kernel, scatter_kernel


BLK = 2000


def _dinv_of(dp):
    return lax.rsqrt(dp[0, :, 0:1] + dp[1, :, 0:1] + 1.0)


def _m1_body(x_ref, w_ref, dp_ref, g_ref):
    dinv = _dinv_of(dp_ref[...])
    g_ref[...] = dinv * jnp.dot(x_ref[...], w_ref[...],
                                preferred_element_type=jnp.float32)


def _m2_body(p_ref, g1_ref, dp_ref, b_ref, w_ref, g2_ref):
    dinv = _dinv_of(dp_ref[...])
    p = p_ref[...]
    z = jnp.maximum(dinv * (p[0] + p[1] + g1_ref[...]) + b_ref[...], 0.0)
    g2_ref[...] = dinv * jnp.dot(z, w_ref[...],
                                 preferred_element_type=jnp.float32)


def _m3_body(p_ref, g2_ref, dp_ref, b_ref, o_ref):
    dinv = _dinv_of(dp_ref[...])
    p = p_ref[...]
    o_ref[...] = dinv * (p[0] + p[1] + g2_ref[...]) + b_ref[...]


def _row_spec(blk=BLK):
    return pl.BlockSpec((blk, D), lambda i: (i, 0))


def _m1_call(x, W1, degp):
    return pl.pallas_call(
        _m1_body,
        grid=(N // BLK,),
        in_specs=[
            _row_spec(),
            pl.BlockSpec((D, D), lambda i: (0, 0)),
            pl.BlockSpec((2, BLK, 16), lambda i: (0, i, 0)),
        ],
        out_specs=_row_spec(),
        out_shape=jax.ShapeDtypeStruct((N, D), jnp.float32),
    )(x, W1, degp)


def _m2_call(P1, g1, degp, b1, W2):
    return pl.pallas_call(
        _m2_body,
        grid=(N // BLK,),
        in_specs=[
            pl.BlockSpec((2, BLK, D), lambda i: (0, i, 0)),
            _row_spec(),
            pl.BlockSpec((2, BLK, 16), lambda i: (0, i, 0)),
            pl.BlockSpec((1, D), lambda i: (0, 0)),
            pl.BlockSpec((D, D), lambda i: (0, 0)),
        ],
        out_specs=_row_spec(),
        out_shape=jax.ShapeDtypeStruct((N, D), jnp.float32),
    )(P1, g1, degp, b1, W2)


def _m3_call(P2, g2, degp, b2):
    return pl.pallas_call(
        _m3_body,
        grid=(N // BLK,),
        in_specs=[
            pl.BlockSpec((2, BLK, D), lambda i: (0, i, 0)),
            _row_spec(),
            pl.BlockSpec((2, BLK, 16), lambda i: (0, i, 0)),
            pl.BlockSpec((1, D), lambda i: (0, 0)),
        ],
        out_specs=_row_spec(),
        out_shape=jax.ShapeDtypeStruct((N, D), jnp.float32),
    )(P2, g2, degp, b2)


def kernel(x, edge_index, W1, b1, W2, b2):
    deg_kernel, scatter_kernel = _sc_kernels()
    src = edge_index[0]
    dst = edge_index[1]
    zerosD = jnp.zeros((WR_LAST, D), jnp.float32)
    onesD = jnp.ones((CH, D), jnp.float32)
    ew = edge_index.reshape(2, NW, EPW)[:, :, :NFULL * CH]
    srcC = ew[0].reshape(NW, NFULL, CH)
    dstC = ew[1].reshape(NW, NFULL, CH)
    idx3 = jnp.stack([srcC, dstC], axis=2).reshape(NW * NFULL, 2, CH)

    degp = deg_kernel(dst, zerosD, onesD).reshape(NC, N, D)[:, :, :16]
    g1 = _m1_call(x, W1, degp)
    P1 = scatter_kernel(g1, idx3, src, dst, zerosD).reshape(NC, N, D)
    g2 = _m2_call(P1, g1, degp, b1.reshape(1, D), W2)
    P2 = scatter_kernel(g2, idx3, src, dst, zerosD).reshape(NC, N, D)
    out = _m3_call(P2, g2, degp, b2.reshape(1, D))
    return out


# speedup vs baseline: 27.0634x; 1.1369x over previous
import functools

import jax
import jax.numpy as jnp
from jax import lax
from jax.experimental import pallas as pl
from jax.experimental.pallas import tpu as pltpu
from jax.experimental.pallas import tpu_sc as plsc

N = 10000
D = 128
E = 320000
NC, NS, NW = 2, 16, 32
CH = 128
NCHUNK = E // CH
NROUND = NCHUNK // NW
NPAIR = NROUND // 2
XTRA = NCHUNK - NROUND * NW
WR = 624
WR_LAST = N - 15 * WR


@functools.cache
def _sc_kernels():
    mesh = plsc.VectorSubcoreMesh(core_axis_name="c", subcore_axis_name="s",
                                  num_cores=NC, num_subcores=NS)

    def _zero_acc(s, zeros_hbm, acc):
        @pl.when(s < NS - 1)
        def _zero():
            pltpu.sync_copy(zeros_hbm.at[pl.ds(0, WR)],
                            acc.at[pl.ds(s * WR, WR)])

        @pl.when(s == NS - 1)
        def _zero_last():
            pltpu.sync_copy(zeros_hbm, acc.at[pl.ds(15 * WR, WR_LAST)])

    def _writeout(c, s, acc, out_hbm):
        @pl.when(s < NS - 1)
        def _wout():
            pltpu.sync_copy(acc.at[pl.ds(s * WR, WR)],
                            out_hbm.at[pl.ds(c * N + s * WR, WR)])

        @pl.when(s == NS - 1)
        def _wout_last():
            pltpu.sync_copy(acc.at[pl.ds(15 * WR, WR_LAST)],
                            out_hbm.at[pl.ds(c * N + 15 * WR, WR_LAST)])

    @functools.partial(
        pl.kernel,
        out_type=jax.ShapeDtypeStruct((NC * N, D), jnp.float32),
        mesh=mesh,
        scratch_types=[
            pltpu.VMEM((2, CH), jnp.int32),
            pltpu.VMEM((2, CH), jnp.int32),
            pltpu.VMEM((CH, D), jnp.float32),
            pltpu.VMEM_SHARED((N, D), jnp.float32),
            pltpu.SemaphoreType.DMA,
            pltpu.SemaphoreType.DMA,
        ],
    )
    def deg_kernel(ei_hbm, zeros_hbm, ones_hbm, out_hbm,
                   ip0, ip1, ones_v, acc, ss0, ss1):
        c = lax.axis_index("c")
        s = lax.axis_index("s")
        w = c * NS + s
        _zero_acc(s, zeros_hbm, acc)
        pltpu.sync_copy(ones_hbm, ones_v)
        plsc.subcore_barrier()

        pltpu.sync_copy(ei_hbm.at[:, pl.ds(w * CH, CH)], ip0)

        @pl.loop(0, NPAIR)
        def _pair(k):
            off = (2 * k * NW + w) * CH
            pltpu.async_copy(ones_v, acc.at[ip0.at[1]], ss0, add=True)

            @pl.when(k > 0)
            def _drain_odd():
                pltpu.make_async_copy(ones_v, acc.at[ip1.at[1]], ss1).wait()

            pltpu.sync_copy(ei_hbm.at[:, pl.ds(off + NW * CH, CH)], ip1)
            pltpu.async_copy(ones_v, acc.at[ip1.at[1]], ss1, add=True)

            @pl.when(k < NPAIR - 1)
            def _prefetch_even():
                pltpu.make_async_copy(ones_v, acc.at[ip0.at[1]], ss0).wait()
                pltpu.sync_copy(ei_hbm.at[:, pl.ds(off + 2 * NW * CH, CH)],
                                ip0)

        pltpu.make_async_copy(ones_v, acc.at[ip0.at[1]], ss0).wait()
        pltpu.make_async_copy(ones_v, acc.at[ip1.at[1]], ss1).wait()

        @pl.when(w < XTRA)
        def _extra():
            pltpu.sync_copy(ei_hbm.at[:, pl.ds((NROUND * NW + w) * CH, CH)],
                            ip0)
            pltpu.sync_copy(ones_v, acc.at[ip0.at[1]], add=True)

        plsc.subcore_barrier()
        _writeout(c, s, acc, out_hbm)

    @functools.partial(
        pl.kernel,
        out_type=jax.ShapeDtypeStruct((NC * N, D), jnp.float32),
        mesh=mesh,
        scratch_types=[
            pltpu.VMEM((2, CH), jnp.int32),
            pltpu.VMEM((2, CH), jnp.int32),
            pltpu.VMEM((CH, D), jnp.float32),
            pltpu.VMEM((CH, D), jnp.float32),
            pltpu.VMEM_SHARED((N, D), jnp.float32),
            pltpu.SemaphoreType.DMA,
            pltpu.SemaphoreType.DMA,
            pltpu.SemaphoreType.DMA,
            pltpu.SemaphoreType.DMA,
        ],
    )
    def scatter_kernel(g_hbm, ei_hbm, zeros_hbm, out_hbm,
                       ip0, ip1, rows0, rows1, acc, gs0, gs1, ss0, ss1):
        c = lax.axis_index("c")
        s = lax.axis_index("s")
        w = c * NS + s
        _zero_acc(s, zeros_hbm, acc)
        plsc.subcore_barrier()

        pltpu.sync_copy(ei_hbm.at[:, pl.ds(w * CH, CH)], ip0)
        pltpu.async_copy(g_hbm.at[ip0.at[0]], rows0, gs0)

        @pl.loop(0, NPAIR)
        def _pair(k):
            off = (2 * k * NW + w) * CH

            @pl.when(k > 0)
            def _drain_prev_odd():
                pltpu.make_async_copy(rows1, acc.at[ip1.at[1]], ss1).wait()

            pltpu.sync_copy(ei_hbm.at[:, pl.ds(off + NW * CH, CH)], ip1)
            pltpu.async_copy(g_hbm.at[ip1.at[0]], rows1, gs1)
            pltpu.make_async_copy(g_hbm.at[ip0.at[0]], rows0, gs0).wait()
            pltpu.async_copy(rows0, acc.at[ip0.at[1]], ss0, add=True)

            @pl.when(k < NPAIR - 1)
            def _prefetch_even():
                pltpu.make_async_copy(rows0, acc.at[ip0.at[1]], ss0).wait()
                pltpu.sync_copy(ei_hbm.at[:, pl.ds(off + 2 * NW * CH, CH)],
                                ip0)
                pltpu.async_copy(g_hbm.at[ip0.at[0]], rows0, gs0)

            pltpu.make_async_copy(g_hbm.at[ip1.at[0]], rows1, gs1).wait()
            pltpu.async_copy(rows1, acc.at[ip1.at[1]], ss1, add=True)

        pltpu.make_async_copy(rows0, acc.at[ip0.at[1]], ss0).wait()
        pltpu.make_async_copy(rows1, acc.at[ip1.at[1]], ss1).wait()

        @pl.when(w < XTRA)
        def _extra():
            pltpu.sync_copy(ei_hbm.at[:, pl.ds((NROUND * NW + w) * CH, CH)],
                            ip0)
            pltpu.async_copy(g_hbm.at[ip0.at[0]], rows0, gs0).wait()
            pltpu.sync_copy(rows0, acc.at[ip0.at[1]], add=True)

        plsc.subcore_barrier()
        _writeout(c, s, acc, out_hbm)

    return deg_kernel, scatter_kernel


BLK = 2000


def _dinv_of(dp):
    return lax.rsqrt(dp[0, :, 0:1] + dp[1, :, 0:1] + 1.0)


def _m1_body(x_ref, w_ref, dp_ref, g_ref):
    dinv = _dinv_of(dp_ref[...])
    g_ref[...] = dinv * jnp.dot(x_ref[...], w_ref[...],
                                preferred_element_type=jnp.float32)


def _m2_body(p_ref, g1_ref, dp_ref, b_ref, w_ref, g2_ref):
    dinv = _dinv_of(dp_ref[...])
    p = p_ref[...]
    z = jnp.maximum(dinv * (p[0] + p[1] + g1_ref[...]) + b_ref[...], 0.0)
    g2_ref[...] = dinv * jnp.dot(z, w_ref[...],
                                 preferred_element_type=jnp.float32)


def _m3_body(p_ref, g2_ref, dp_ref, b_ref, o_ref):
    dinv = _dinv_of(dp_ref[...])
    p = p_ref[...]
    o_ref[...] = dinv * (p[0] + p[1] + g2_ref[...]) + b_ref[...]


def _row_spec(blk=BLK):
    return pl.BlockSpec((blk, D), lambda i: (i, 0))


def _m1_call(x, W1, degp):
    return pl.pallas_call(
        _m1_body,
        grid=(N // BLK,),
        in_specs=[
            _row_spec(),
            pl.BlockSpec((D, D), lambda i: (0, 0)),
            pl.BlockSpec((2, BLK, 16), lambda i: (0, i, 0)),
        ],
        out_specs=_row_spec(),
        out_shape=jax.ShapeDtypeStruct((N, D), jnp.float32),
    )(x, W1, degp)


def _m2_call(P1, g1, degp, b1, W2):
    return pl.pallas_call(
        _m2_body,
        grid=(N // BLK,),
        in_specs=[
            pl.BlockSpec((2, BLK, D), lambda i: (0, i, 0)),
            _row_spec(),
            pl.BlockSpec((2, BLK, 16), lambda i: (0, i, 0)),
            pl.BlockSpec((1, D), lambda i: (0, 0)),
            pl.BlockSpec((D, D), lambda i: (0, 0)),
        ],
        out_specs=_row_spec(),
        out_shape=jax.ShapeDtypeStruct((N, D), jnp.float32),
    )(P1, g1, degp, b1, W2)


def _m3_call(P2, g2, degp, b2):
    return pl.pallas_call(
        _m3_body,
        grid=(N // BLK,),
        in_specs=[
            pl.BlockSpec((2, BLK, D), lambda i: (0, i, 0)),
            _row_spec(),
            pl.BlockSpec((2, BLK, 16), lambda i: (0, i, 0)),
            pl.BlockSpec((1, D), lambda i: (0, 0)),
        ],
        out_specs=_row_spec(),
        out_shape=jax.ShapeDtypeStruct((N, D), jnp.float32),
    )(P2, g2, degp, b2)


def kernel(x, edge_index, W1, b1, W2, b2):
    deg_kernel, scatter_kernel = _sc_kernels()
    zerosD = jnp.zeros((WR_LAST, D), jnp.float32)
    onesD = jnp.ones((CH, D), jnp.float32)

    degp = deg_kernel(edge_index, zerosD, onesD).reshape(NC, N, D)[:, :, :16]
    g1 = _m1_call(x, W1, degp)
    P1 = scatter_kernel(g1, edge_index, zerosD).reshape(NC, N, D)
    g2 = _m2_call(P1, g1, degp, b1.reshape(1, D), W2)
    P2 = scatter_kernel(g2, edge_index, zerosD).reshape(NC, N, D)
    out = _m3_call(P2, g2, degp, b2.reshape(1, D))
    return out


# speedup vs baseline: 27.0773x; 1.0005x over previous
import functools

import jax
import jax.numpy as jnp
from jax import lax
from jax.experimental import pallas as pl
from jax.experimental.pallas import tpu as pltpu
from jax.experimental.pallas import tpu_sc as plsc

N = 10000
D = 128
E = 320000
NC, NS, NW = 2, 16, 32
CH = 128
NCHUNK = E // CH
NROUND = NCHUNK // NW
NPAIR = NROUND // 2
XTRA = NCHUNK - NROUND * NW
WR = 624
WR_LAST = N - 15 * WR


@functools.cache
def _sc_kernels():
    mesh = plsc.VectorSubcoreMesh(core_axis_name="c", subcore_axis_name="s",
                                  num_cores=NC, num_subcores=NS)

    def _zero_acc(s, zeros_hbm, acc):
        @pl.when(s < NS - 1)
        def _zero():
            pltpu.sync_copy(zeros_hbm.at[pl.ds(0, WR)],
                            acc.at[pl.ds(s * WR, WR)])

        @pl.when(s == NS - 1)
        def _zero_last():
            pltpu.sync_copy(zeros_hbm, acc.at[pl.ds(15 * WR, WR_LAST)])

    def _writeout(c, s, acc, out_hbm):
        @pl.when(s < NS - 1)
        def _wout():
            pltpu.sync_copy(acc.at[pl.ds(s * WR, WR)],
                            out_hbm.at[pl.ds(c * N + s * WR, WR)])

        @pl.when(s == NS - 1)
        def _wout_last():
            pltpu.sync_copy(acc.at[pl.ds(15 * WR, WR_LAST)],
                            out_hbm.at[pl.ds(c * N + 15 * WR, WR_LAST)])

    @functools.partial(
        pl.kernel,
        out_type=jax.ShapeDtypeStruct((NC * N, D), jnp.float32),
        mesh=mesh,
        scratch_types=[
            pltpu.VMEM((2, CH), jnp.int32),
            pltpu.VMEM((2, CH), jnp.int32),
            pltpu.VMEM((CH, D), jnp.float32),
            pltpu.VMEM_SHARED((N, D), jnp.float32),
            pltpu.SemaphoreType.DMA,
            pltpu.SemaphoreType.DMA,
        ],
    )
    def deg_kernel(ei_hbm, zeros_hbm, ones_hbm, out_hbm,
                   ip0, ip1, ones_v, acc, ss0, ss1):
        c = lax.axis_index("c")
        s = lax.axis_index("s")
        w = c * NS + s
        _zero_acc(s, zeros_hbm, acc)
        pltpu.sync_copy(ones_hbm, ones_v)
        plsc.subcore_barrier()

        pltpu.sync_copy(ei_hbm.at[:, pl.ds(w * CH, CH)], ip0)

        @pl.loop(0, NPAIR)
        def _pair(k):
            off = (2 * k * NW + w) * CH
            pltpu.async_copy(ones_v, acc.at[ip0.at[1]], ss0, add=True)

            @pl.when(k > 0)
            def _drain_odd():
                pltpu.make_async_copy(ones_v, acc.at[ip1.at[1]], ss1).wait()

            pltpu.sync_copy(ei_hbm.at[:, pl.ds(off + NW * CH, CH)], ip1)
            pltpu.async_copy(ones_v, acc.at[ip1.at[1]], ss1, add=True)

            @pl.when(k < NPAIR - 1)
            def _prefetch_even():
                pltpu.make_async_copy(ones_v, acc.at[ip0.at[1]], ss0).wait()
                pltpu.sync_copy(ei_hbm.at[:, pl.ds(off + 2 * NW * CH, CH)],
                                ip0)

        pltpu.make_async_copy(ones_v, acc.at[ip0.at[1]], ss0).wait()
        pltpu.make_async_copy(ones_v, acc.at[ip1.at[1]], ss1).wait()

        @pl.when(w < XTRA)
        def _extra():
            pltpu.sync_copy(ei_hbm.at[:, pl.ds((NROUND * NW + w) * CH, CH)],
                            ip0)
            pltpu.sync_copy(ones_v, acc.at[ip0.at[1]], add=True)

        plsc.subcore_barrier()
        _writeout(c, s, acc, out_hbm)

    @functools.partial(
        pl.kernel,
        out_type=jax.ShapeDtypeStruct((NC * N, D), jnp.float32),
        mesh=mesh,
        scratch_types=[
            pltpu.VMEM((2, CH), jnp.int32),
            pltpu.VMEM((2, CH), jnp.int32),
            pltpu.VMEM((CH, D), jnp.float32),
            pltpu.VMEM((CH, D), jnp.float32),
            pltpu.VMEM_SHARED((N, D), jnp.float32),
            pltpu.SemaphoreType.DMA,
            pltpu.SemaphoreType.DMA,
            pltpu.SemaphoreType.DMA,
            pltpu.SemaphoreType.DMA,
        ],
    )
    def scatter_kernel(g_hbm, ei_hbm, zeros_hbm, out_hbm,
                       ip0, ip1, rows0, rows1, acc, gs0, gs1, ss0, ss1):
        c = lax.axis_index("c")
        s = lax.axis_index("s")
        w = c * NS + s
        _zero_acc(s, zeros_hbm, acc)
        plsc.subcore_barrier()

        pltpu.sync_copy(ei_hbm.at[:, pl.ds(w * CH, CH)], ip0)
        pltpu.async_copy(g_hbm.at[ip0.at[0]], rows0, gs0)

        @pl.loop(0, NPAIR)
        def _pair(k):
            off = (2 * k * NW + w) * CH

            @pl.when(k > 0)
            def _drain_prev_odd():
                pltpu.make_async_copy(rows1, acc.at[ip1.at[1]], ss1).wait()

            pltpu.sync_copy(ei_hbm.at[:, pl.ds(off + NW * CH, CH)], ip1)
            pltpu.async_copy(g_hbm.at[ip1.at[0]], rows1, gs1)
            pltpu.make_async_copy(g_hbm.at[ip0.at[0]], rows0, gs0).wait()
            pltpu.async_copy(rows0, acc.at[ip0.at[1]], ss0, add=True)

            @pl.when(k < NPAIR - 1)
            def _prefetch_even():
                pltpu.make_async_copy(rows0, acc.at[ip0.at[1]], ss0).wait()
                pltpu.sync_copy(ei_hbm.at[:, pl.ds(off + 2 * NW * CH, CH)],
                                ip0)
                pltpu.async_copy(g_hbm.at[ip0.at[0]], rows0, gs0)

            pltpu.make_async_copy(g_hbm.at[ip1.at[0]], rows1, gs1).wait()
            pltpu.async_copy(rows1, acc.at[ip1.at[1]], ss1, add=True)

        pltpu.make_async_copy(rows0, acc.at[ip0.at[1]], ss0).wait()
        pltpu.make_async_copy(rows1, acc.at[ip1.at[1]], ss1).wait()

        @pl.when(w < XTRA)
        def _extra():
            pltpu.sync_copy(ei_hbm.at[:, pl.ds((NROUND * NW + w) * CH, CH)],
                            ip0)
            pltpu.async_copy(g_hbm.at[ip0.at[0]], rows0, gs0).wait()
            pltpu.sync_copy(rows0, acc.at[ip0.at[1]], add=True)

        plsc.subcore_barrier()
        _writeout(c, s, acc, out_hbm)

    return deg_kernel, scatter_kernel


BLK = 2000


def _dinv_of(dp):
    return lax.rsqrt(dp[0, :, 0:1] + dp[1, :, 0:1] + 1.0)


def _m1_body(x_ref, w_ref, dp_ref, g_ref):
    dinv = _dinv_of(dp_ref[...])
    g_ref[...] = dinv * jnp.dot(x_ref[...], w_ref[...],
                                preferred_element_type=jnp.float32)


def _m2_body(p_ref, g1_ref, dp_ref, b_ref, w_ref, g2_ref):
    dinv = _dinv_of(dp_ref[...])
    p = p_ref[...]
    z = jnp.maximum(dinv * (p[0] + p[1] + g1_ref[...]) + b_ref[...], 0.0)
    g2_ref[...] = dinv * jnp.dot(z, w_ref[...],
                                 preferred_element_type=jnp.float32)


def _m3_body(p_ref, g2_ref, dp_ref, b_ref, o_ref):
    dinv = _dinv_of(dp_ref[...])
    p = p_ref[...]
    o_ref[...] = dinv * (p[0] + p[1] + g2_ref[...]) + b_ref[...]


def _row_spec(blk=BLK):
    return pl.BlockSpec((blk, D), lambda i: (i, 0))


def _m1_call(x, W1, degp):
    return pl.pallas_call(
        _m1_body,
        grid=(N // BLK,),
        in_specs=[
            _row_spec(),
            pl.BlockSpec((D, D), lambda i: (0, 0)),
            pl.BlockSpec((2, BLK, D), lambda i: (0, i, 0)),
        ],
        out_specs=_row_spec(),
        out_shape=jax.ShapeDtypeStruct((N, D), jnp.float32),
    )(x, W1, degp)


def _m2_call(P1, g1, degp, b1, W2):
    return pl.pallas_call(
        _m2_body,
        grid=(N // BLK,),
        in_specs=[
            pl.BlockSpec((2, BLK, D), lambda i: (0, i, 0)),
            _row_spec(),
            pl.BlockSpec((2, BLK, D), lambda i: (0, i, 0)),
            pl.BlockSpec((1, D), lambda i: (0, 0)),
            pl.BlockSpec((D, D), lambda i: (0, 0)),
        ],
        out_specs=_row_spec(),
        out_shape=jax.ShapeDtypeStruct((N, D), jnp.float32),
    )(P1, g1, degp, b1, W2)


def _m3_call(P2, g2, degp, b2):
    return pl.pallas_call(
        _m3_body,
        grid=(N // BLK,),
        in_specs=[
            pl.BlockSpec((2, BLK, D), lambda i: (0, i, 0)),
            _row_spec(),
            pl.BlockSpec((2, BLK, D), lambda i: (0, i, 0)),
            pl.BlockSpec((1, D), lambda i: (0, 0)),
        ],
        out_specs=_row_spec(),
        out_shape=jax.ShapeDtypeStruct((N, D), jnp.float32),
    )(P2, g2, degp, b2)


def kernel(x, edge_index, W1, b1, W2, b2):
    deg_kernel, scatter_kernel = _sc_kernels()
    zerosD = jnp.zeros((WR_LAST, D), jnp.float32)
    onesD = jnp.ones((CH, D), jnp.float32)

    degp = deg_kernel(edge_index, zerosD, onesD).reshape(NC, N, D)
    g1 = _m1_call(x, W1, degp)
    P1 = scatter_kernel(g1, edge_index, zerosD).reshape(NC, N, D)
    g2 = _m2_call(P1, g1, degp, b1.reshape(1, D), W2)
    P2 = scatter_kernel(g2, edge_index, zerosD).reshape(NC, N, D)
    out = _m3_call(P2, g2, degp, b2.reshape(1, D))
    return out
